# Initial kernel scaffold; baseline (speedup 1.0000x reference)
import jax
import jax.numpy as jnp
from jax.experimental import pallas as pl


def kernel(g, idx):
    raise NotImplementedError("write your pallas kernel here")


# speedup vs baseline: 8.1727x; 8.1727x over previous
import functools

import jax
import jax.numpy as jnp
from jax import lax
from jax.experimental import pallas as pl
from jax.experimental.pallas import tpu as pltpu
from jax.experimental.pallas import tpu_sc as plsc

NUM_DAGS = 100000
P = 32
D = P * P
BATCH = 16384

NC = 2
NS = 16
NW = NC * NS
B_PER_W = BATCH // NW
C = 32
NCHUNK = B_PER_W // C
NBUF = 3

_mesh = plsc.VectorSubcoreMesh(core_axis_name="c", subcore_axis_name="s")


@functools.partial(
    pl.kernel,
    mesh=_mesh,
    out_type=jax.ShapeDtypeStruct((BATCH, D), jnp.float32),
    compiler_params=pltpu.CompilerParams(needs_layout_passes=False),
    scratch_types=(
        [pltpu.VMEM((NCHUNK, C), jnp.int32)]
        + [pltpu.VMEM((C, D), jnp.float32) for _ in range(NBUF)]
        + [pltpu.SemaphoreType.DMA for _ in range(2 * NBUF)]
    ),
)
def _gather_masked(g_hbm, idx_hbm, out_hbm, idx_v, *bufs_and_sems):
    bufs = bufs_and_sems[:NBUF]
    gsem = bufs_and_sems[NBUF : 2 * NBUF]
    osem = bufs_and_sems[2 * NBUF :]

    wid = lax.axis_index("s") * NC + lax.axis_index("c")
    row0 = wid * B_PER_W

    pltpu.sync_copy(idx_hbm.at[wid], idx_v)

    zeros16 = jnp.zeros((16,), jnp.float32)
    iota16 = lax.iota(jnp.int32, 16)

    def zero_diag(buf):
        def zbody(gidx, carry):
            d = gidx * 16 + iota16
            rows = lax.shift_right_logical(d, 5)
            cols = 33 * (d & 31)
            plsc.store_scatter(buf, [rows, cols], zeros16)
            return carry
        lax.fori_loop(0, 2 * C, zbody, 0)

    def start_gather(j):
        b = j % NBUF
        return pltpu.async_copy(g_hbm.at[idx_v.at[j]], bufs[b], gsem[b])

    gathers = [None] * NBUF
    outs = [None] * NBUF

    gathers[0] = start_gather(0)
    for j in range(NCHUNK):
        b = j % NBUF
        nxt = j + 1
        if nxt < NCHUNK:
            nb = nxt % NBUF
            if outs[nb] is not None:
                outs[nb].wait()
                outs[nb] = None
            gathers[nb] = start_gather(nxt)
        gathers[b].wait()
        zero_diag(bufs[b])
        outs[b] = pltpu.async_copy(
            bufs[b], out_hbm.at[pl.ds(row0 + j * C, C)], osem[b]
        )
    for o in outs:
        if o is not None:
            o.wait()


def kernel(g, idx):
    g2 = g.reshape(NUM_DAGS, D)
    idx3 = idx.astype(jnp.int32).reshape(NW, NCHUNK, C)
    out = _gather_masked(g2, idx3)
    return out.reshape(BATCH, P, P)
